# baseline (speedup 1.0000x reference)
import functools

import jax
import jax.numpy as jnp
from jax import lax
from jax.experimental import pallas as pl
from jax.experimental.pallas import tpu as pltpu
from jax.experimental.pallas import tpu_sc as plsc

L = 16
NC, NS = 2, 16
NW = NC * NS
SL = 1280
CC = SL // L


def _sc_loss(raw_hbm, preds8_hbm, tgt_hbm, len_hbm, out_hbm,
             raw_v, soa_v, tgt_v, len_v, resm_v, resi_v, shrm, shri,
             mrgm_v, mrgi_v, row_v, out_v, scr_v):
    B = preds8_hbm.shape[0]
    T = tgt_hbm.shape[0] // B
    NG = resm_v.shape[0]
    bpc = B // NC

    c = lax.axis_index("c")
    s = lax.axis_index("s")
    wid = s * NC + c

    pltpu.sync_copy(tgt_hbm, tgt_v)
    pltpu.sync_copy(len_hbm, len_v)

    iota = lax.iota(jnp.int32, L)
    lv = len_v[...]
    base_idx = jnp.full((L,), s * SL) + iota

    lanes = [(5 * iota + r) & 15 for r in range(5)]
    masks = [[lax.shift_right_logical(5 * iota + r, 4) == q for q in range(5)]
             for r in range(5)]

    for i in range(bpc):
        b = 2 * i + c
        len_b = jnp.where(c == 0, lv[2 * i], lv[2 * i + 1])
        pltpu.sync_copy(raw_hbm.at[b, s], raw_v)

        def convert(j, _):
            rvs = [raw_v[5 * j + q] for q in range(5)]

            def field(r):
                x = jnp.zeros((L,), jnp.float32)
                for q in range(5):
                    x = jnp.where(masks[r][q], rvs[q][lanes[r]], x)
                return x

            cx, cy, w, h = field(1), field(2), field(3), field(4)
            hw = w * 0.5
            hh = h * 0.5
            x1 = cx - hw
            x2 = cx + hw
            y1 = cy - hh
            y2 = cy + hh
            soa_v[0, j] = x1
            soa_v[1, j] = x2
            soa_v[2, j] = y1
            soa_v[3, j] = y2
            soa_v[4, j] = (x2 - x1) * (y2 - y1)
            return 0

        lax.fori_loop(0, CC, convert, 0, unroll=2)

        def per_target(t, carry):
            g = b * T + t
            trow = tgt_v[g]
            s_cx, s_cy, s_w, s_h = trow[1], trow[2], trow[3], trow[4]
            t_x1 = jnp.full((L,), s_cx - s_w * 0.5)
            t_y1 = jnp.full((L,), s_cy - s_h * 0.5)
            t_x2 = jnp.full((L,), s_cx + s_w * 0.5)
            t_y2 = jnp.full((L,), s_cy + s_h * 0.5)
            area_t = (t_x2 - t_x1) * (t_y2 - t_y1)

            def per_chunk(j, carry):
                run_max, run_idx, idxs = carry
                x1 = soa_v[0, j]
                x2 = soa_v[1, j]
                y1 = soa_v[2, j]
                y2 = soa_v[3, j]
                ap = soa_v[4, j]
                ix1 = jnp.maximum(x1, t_x1)
                iy1 = jnp.maximum(y1, t_y1)
                ix2 = jnp.minimum(x2, t_x2)
                iy2 = jnp.minimum(y2, t_y2)
                inter = jnp.maximum(0.0, ix2 - ix1) * jnp.maximum(0.0, iy2 - iy1)
                iou = inter / (ap + area_t - inter + 1e-06)
                upd = iou > run_max
                run_max = jnp.where(upd, iou, run_max)
                run_idx = jnp.where(upd, idxs, run_idx)
                return run_max, run_idx, idxs + L

            init = (jnp.full((L,), -jnp.inf, jnp.float32),
                    jnp.zeros((L,), jnp.int32), base_idx)
            run_max, run_idx, _ = lax.fori_loop(0, CC, per_chunk, init, unroll=4)

            m = run_max
            for sh in (8, 4, 2, 1):
                m = jnp.maximum(m, m[iota ^ sh])
            cand = jnp.where(run_max == m, run_idx.astype(jnp.float32),
                             jnp.full((L,), 1e9, jnp.float32))
            for sh in (8, 4, 2, 1):
                cand = jnp.minimum(cand, cand[iota ^ sh])

            grp = lax.shift_right_logical(g, 4)
            posv = jnp.full((L,), g & (L - 1))
            resm_v[grp] = jnp.where(iota == posv, m, resm_v[grp])
            resi_v[grp] = jnp.where(iota == posv, cand, resi_v[grp])
            return carry

        lax.fori_loop(0, len_b, per_target, 0)

    pltpu.sync_copy(resm_v, shrm.at[s])
    pltpu.sync_copy(resi_v, shri.at[s])
    plsc.subcore_barrier()
    pltpu.sync_copy(shrm, mrgm_v)
    pltpu.sync_copy(shri, mrgi_v)

    out_v[...] = jnp.zeros((L,), jnp.float32)
    for gi in range(2):
        grp = s + NS * gi

        @pl.when(grp < NG)
        def _():
            mm = jnp.full((L,), -jnp.inf, jnp.float32)
            mi = jnp.zeros((L,), jnp.float32)
            for tt in range(NS):
                vm = mrgm_v[tt, grp]
                vi = mrgi_v[tt, grp]
                upd = vm > mm
                mm = jnp.where(upd, vm, mm)
                mi = jnp.where(upd, vi, mi)
            for po in range(L):
                g = grp * L + po
                b = g // T
                t = g - T * b
                scr_v[...] = lv[jnp.full((L,), b)]
                len_b = scr_v[...][0]
                valid = (t < len_b) & ((b & 1) == c)

                @pl.when(valid)
                def _():
                    best = jnp.where(mm[po] > 0.0, mi[po], 0.0).astype(jnp.int32)
                    pltpu.sync_copy(preds8_hbm.at[b, best], row_v.at[pl.ds(0, 8)])
                    d = row_v[...] - tgt_v[g]
                    out_v[...] = out_v[...] + jnp.where(iota < 5, d * d, 0.0)

    pltpu.sync_copy(out_v, out_hbm.at[wid])


def kernel(predictions, targets, lengths):
    B, N, F = predictions.shape
    T = targets.shape[1]
    NPAD = NS * SL
    NG = (B * T + L - 1) // L
    preds_pad = jnp.pad(predictions, ((0, 0), (0, NPAD - N), (0, 0)),
                        constant_values=1e30)
    raw = jnp.reshape(preds_pad, (B, NS, SL * F // L, L))
    preds8 = jnp.pad(predictions, ((0, 0), (0, 0), (0, 8 - F)))
    tgt_pad = jnp.reshape(jnp.pad(targets, ((0, 0), (0, 0), (0, L - F))),
                          (B * T, L))
    len_pad = jnp.pad(lengths.astype(jnp.int32), (0, L - B))

    mesh = plsc.VectorSubcoreMesh(core_axis_name="c", subcore_axis_name="s",
                                  num_cores=NC, num_subcores=NS)
    run = pl.kernel(
        _sc_loss,
        out_type=jax.ShapeDtypeStruct((NW, L), jnp.float32),
        mesh=mesh,
        compiler_params=pltpu.CompilerParams(use_tc_tiling_on_sc=False),
        scratch_types=[
            pltpu.VMEM((SL * F // L, L), jnp.float32),
            pltpu.VMEM((F, CC, L), jnp.float32),
            pltpu.VMEM((B * T, L), jnp.float32),
            pltpu.VMEM((L,), jnp.int32),
            pltpu.VMEM((NG, L), jnp.float32),
            pltpu.VMEM((NG, L), jnp.float32),
            pltpu.VMEM_SHARED((NS, NG, L), jnp.float32),
            pltpu.VMEM_SHARED((NS, NG, L), jnp.float32),
            pltpu.VMEM((NS, NG, L), jnp.float32),
            pltpu.VMEM((NS, NG, L), jnp.float32),
            pltpu.VMEM((L,), jnp.float32),
            pltpu.VMEM((L,), jnp.float32),
            pltpu.VMEM((L,), jnp.int32),
        ],
    )
    partial = run(raw, preds8, tgt_pad, len_pad)
    return jnp.sum(partial) / B


# speedup vs baseline: 1.8771x; 1.8771x over previous
import functools

import jax
import jax.numpy as jnp
from jax import lax
from jax.experimental import pallas as pl
from jax.experimental.pallas import tpu as pltpu
from jax.experimental.pallas import tpu_sc as plsc

L = 16
NC, NS = 2, 16
NW = NC * NS
SL = 1248
VPS = SL * 5 // L


def _sc_loss(B, T, raw_hbm, tgt_hbm, len_hbm, out_hbm,
             raw_v, soa_v, tgt_v, len_v, resm_v, resi_v, shrm, shri,
             mrgm_v, mrgi_v, row_v, out_v, scr_v):
    NG = resm_v.shape[0]
    bpc = B // NC

    c = lax.axis_index("c")
    s = lax.axis_index("s")
    wid = s * NC + c

    pltpu.sync_copy(tgt_hbm, tgt_v)
    pltpu.sync_copy(len_hbm, len_v)

    iota = lax.iota(jnp.int32, L)
    lv = len_v[...]
    base_idx = jnp.full((L,), s * SL) + iota
    CCH = SL // L + 2

    lanes = [(5 * iota + r) & 15 for r in range(5)]
    masks = [[lax.shift_right_logical(5 * iota + r, 4) == q for q in range(5)]
             for r in range(5)]

    for i in range(bpc):
        b = 2 * i + c
        len_b = jnp.where(c == 0, lv[2 * i], lv[2 * i + 1])
        pltpu.sync_copy(raw_hbm.at[b, pl.ds(s * VPS, VPS)], raw_v.at[pl.ds(0, VPS)])

        @pl.when(s == NS - 1)
        def _():
            pltpu.sync_copy(raw_hbm.at[b, pl.ds(NS * VPS, 10)],
                            raw_v.at[pl.ds(VPS, 10)])

        def convert(j, _):
            rvs = [raw_v[5 * j + q] for q in range(5)]

            def field(r):
                x = jnp.zeros((L,), jnp.float32)
                for q in range(5):
                    x = jnp.where(masks[r][q], rvs[q][lanes[r]], x)
                return x

            cx, cy, w, h = field(1), field(2), field(3), field(4)
            hw = w * 0.5
            hh = h * 0.5
            x1 = cx - hw
            x2 = cx + hw
            y1 = cy - hh
            y2 = cy + hh
            soa_v[0, j] = x1
            soa_v[1, j] = x2
            soa_v[2, j] = y1
            soa_v[3, j] = y2
            soa_v[4, j] = (x2 - x1) * (y2 - y1)
            return 0

        lax.fori_loop(0, CCH, convert, 0, unroll=2)

        @pl.when(s != NS - 1)
        def _():
            pinf = jnp.full((L,), jnp.inf, jnp.float32)
            ninf = jnp.full((L,), -jnp.inf, jnp.float32)
            for j2 in (CCH - 2, CCH - 1):
                soa_v[0, j2] = pinf
                soa_v[1, j2] = ninf
                soa_v[2, j2] = pinf
                soa_v[3, j2] = ninf
                soa_v[4, j2] = jnp.zeros((L,), jnp.float32)

        def per_target(t, carry):
            g = b * T + t
            trow = tgt_v[g]
            s_cx, s_cy, s_w, s_h = trow[1], trow[2], trow[3], trow[4]
            t_x1 = jnp.full((L,), s_cx - s_w * 0.5)
            t_y1 = jnp.full((L,), s_cy - s_h * 0.5)
            t_x2 = jnp.full((L,), s_cx + s_w * 0.5)
            t_y2 = jnp.full((L,), s_cy + s_h * 0.5)
            area_t = (t_x2 - t_x1) * (t_y2 - t_y1)

            def per_chunk(j, carry):
                run_max, run_idx, idxs = carry
                x1 = soa_v[0, j]
                x2 = soa_v[1, j]
                y1 = soa_v[2, j]
                y2 = soa_v[3, j]
                ap = soa_v[4, j]
                ix1 = jnp.maximum(x1, t_x1)
                iy1 = jnp.maximum(y1, t_y1)
                ix2 = jnp.minimum(x2, t_x2)
                iy2 = jnp.minimum(y2, t_y2)
                inter = jnp.maximum(0.0, ix2 - ix1) * jnp.maximum(0.0, iy2 - iy1)
                iou = inter / (ap + area_t - inter + 1e-06)
                upd = iou > run_max
                run_max = jnp.where(upd, iou, run_max)
                run_idx = jnp.where(upd, idxs, run_idx)
                return run_max, run_idx, idxs + L

            init = (jnp.full((L,), -jnp.inf, jnp.float32),
                    jnp.zeros((L,), jnp.int32), base_idx)
            run_max, run_idx, _ = lax.fori_loop(0, CCH, per_chunk, init, unroll=4)

            m = run_max
            for sh in (8, 4, 2, 1):
                m = jnp.maximum(m, m[iota ^ sh])
            cand = jnp.where(run_max == m, run_idx.astype(jnp.float32),
                             jnp.full((L,), 1e9, jnp.float32))
            for sh in (8, 4, 2, 1):
                cand = jnp.minimum(cand, cand[iota ^ sh])

            grp = lax.shift_right_logical(g, 4)
            posv = jnp.full((L,), g & (L - 1))
            resm_v[grp] = jnp.where(iota == posv, m, resm_v[grp])
            resi_v[grp] = jnp.where(iota == posv, cand, resi_v[grp])
            return carry

        lax.fori_loop(0, len_b, per_target, 0)

    pltpu.sync_copy(resm_v, shrm.at[s])
    pltpu.sync_copy(resi_v, shri.at[s])
    plsc.subcore_barrier()
    pltpu.sync_copy(shrm, mrgm_v)
    pltpu.sync_copy(shri, mrgi_v)

    out_v[...] = jnp.zeros((L,), jnp.float32)
    lane0 = iota == 0
    for gi in range(2):
        grp = s + NS * gi

        @pl.when(grp < NG)
        def _():
            mm = jnp.full((L,), -jnp.inf, jnp.float32)
            mi = jnp.zeros((L,), jnp.float32)
            for tt in range(NS):
                vm = mrgm_v[tt, grp]
                vi = mrgi_v[tt, grp]
                upd = vm > mm
                mm = jnp.where(upd, vm, mm)
                mi = jnp.where(upd, vi, mi)
            for po in range(L):
                g = grp * L + po
                b = g // T
                t = g - T * b
                scr_v[...] = lv[jnp.full((L,), b)]
                len_b = scr_v[...][0]
                valid = (t < len_b) & ((b & 1) == c)

                @pl.when(valid)
                def _():
                    best = jnp.where(mm[po] > 0.0, mi[po], 0.0).astype(jnp.int32)
                    vstart = lax.shift_right_logical(best, 4) * 5
                    pltpu.sync_copy(raw_hbm.at[b, pl.ds(vstart, 5)], row_v)
                    off5 = (best & (L - 1)) * 5
                    rv = [row_v[q] for q in range(5)]
                    trow = tgt_v[g]
                    acc = out_v[...]
                    for r in range(5):
                        pos = off5 + r
                        qv = jnp.full((L,), lax.shift_right_logical(pos, 4))
                        lane = jnp.full((L,), pos & (L - 1))
                        val = rv[4][lane]
                        for q2 in range(4):
                            val = jnp.where(qv == q2, rv[q2][lane], val)
                        d = val - jnp.full((L,), trow[r])
                        acc = acc + jnp.where(lane0, d * d, 0.0)
                    out_v[...] = acc

    pltpu.sync_copy(out_v, out_hbm.at[wid])


def kernel(predictions, targets, lengths):
    B, N, F = predictions.shape
    T = targets.shape[1]
    NG = (B * T + L - 1) // L
    raw = jnp.reshape(predictions, (B, N * F // L, L))
    tgt_pad = jnp.reshape(jnp.pad(targets, ((0, 0), (0, 0), (0, L - F))),
                          (B * T, L))
    len_pad = jnp.pad(lengths.astype(jnp.int32), (0, L - B))

    mesh = plsc.VectorSubcoreMesh(core_axis_name="c", subcore_axis_name="s",
                                  num_cores=NC, num_subcores=NS)
    run = pl.kernel(
        functools.partial(_sc_loss, B, T),
        out_type=jax.ShapeDtypeStruct((NW, L), jnp.float32),
        mesh=mesh,
        compiler_params=pltpu.CompilerParams(use_tc_tiling_on_sc=False),
        scratch_types=[
            pltpu.VMEM((VPS + 10, L), jnp.float32),
            pltpu.VMEM((F, SL // L + 2, L), jnp.float32),
            pltpu.VMEM((B * T, L), jnp.float32),
            pltpu.VMEM((L,), jnp.int32),
            pltpu.VMEM((NG, L), jnp.float32),
            pltpu.VMEM((NG, L), jnp.float32),
            pltpu.VMEM_SHARED((NS, NG, L), jnp.float32),
            pltpu.VMEM_SHARED((NS, NG, L), jnp.float32),
            pltpu.VMEM((NS, NG, L), jnp.float32),
            pltpu.VMEM((NS, NG, L), jnp.float32),
            pltpu.VMEM((5, L), jnp.float32),
            pltpu.VMEM((L,), jnp.float32),
            pltpu.VMEM((L,), jnp.int32),
        ],
    )
    partial = run(raw, tgt_pad, len_pad)
    return jnp.sum(partial) / B


# speedup vs baseline: 4.3904x; 2.3388x over previous
import functools

import jax
import jax.numpy as jnp
from jax import lax
from jax.experimental import pallas as pl
from jax.experimental.pallas import tpu as pltpu
from jax.experimental.pallas import tpu_sc as plsc

L = 16
NC, NS = 2, 16
NW = NC * NS
NPAD = 20480
SL = NPAD // NS
RPF = NPAD // 128
RPT = SL // 128
NF = 6


def _tc_relayout(n, nb, xp_ref, out_ref):
    for b in range(nb):
        obj = xp_ref[0, pl.ds(b, 1), :]
        cx = xp_ref[1, pl.ds(b, 1), :]
        cy = xp_ref[2, pl.ds(b, 1), :]
        w = xp_ref[3, pl.ds(b, 1), :]
        h = xp_ref[4, pl.ds(b, 1), :]
        hw = w * 0.5
        hh = h * 0.5
        x1 = cx - hw
        x2 = cx + hw
        y1 = cy - hh
        y2 = cy + hh
        area = (x2 - x1) * (y2 - y1)
        fields = [(obj, 0.0), (x1, jnp.inf), (x2, -jnp.inf), (y1, jnp.inf),
                  (y2, -jnp.inf), (area, 0.0)]
        for f, (vec, padval) in enumerate(fields):
            full = jnp.concatenate(
                [vec, jnp.full((1, NPAD - n), padval, jnp.float32)], axis=1)
            out_ref[pl.ds((b * NF + f) * RPF, RPF), :] = jnp.reshape(
                full, (RPF, 128))


def _sc_loss(B, T, relay_hbm, tgt_hbm, len_hbm, out_hbm,
             soa_v, tgt_v, len_v, resm_v, resi_v, shrm, shri,
             mrgm_v, mrgi_v, row5_v, out_v, scr_v, sem):
    NG = resm_v.shape[0]
    bpc = B // NC

    c = lax.axis_index("c")
    s = lax.axis_index("s")
    wid = s * NC + c

    pltpu.sync_copy(tgt_hbm, tgt_v)
    pltpu.sync_copy(len_hbm, len_v)

    iota = lax.iota(jnp.int32, L)
    lv = len_v[...]
    base_idx = jnp.full((L,), s * SL) + iota

    for i in range(bpc):
        b = 2 * i + c
        len_b = jnp.where(c == 0, lv[2 * i], lv[2 * i + 1])
        cps = [pltpu.async_copy(
                   relay_hbm.at[pl.ds(pl.multiple_of(
                       ((b * NF + f) * RPF + s * RPT) * 128, 8), RPT * 128)],
                   soa_v.at[f], sem)
               for f in range(NF)]
        for cp in cps:
            cp.wait()

        def per_target(t, carry):
            g = b * T + t
            trow = tgt_v[g]
            s_cx, s_cy, s_w, s_h = trow[1], trow[2], trow[3], trow[4]
            t_x1 = jnp.full((L,), s_cx - s_w * 0.5)
            t_y1 = jnp.full((L,), s_cy - s_h * 0.5)
            t_x2 = jnp.full((L,), s_cx + s_w * 0.5)
            t_y2 = jnp.full((L,), s_cy + s_h * 0.5)
            area_t = (t_x2 - t_x1) * (t_y2 - t_y1)

            def per_row(r10, carry):
                run_max, run_idx, idxs = carry
                for k in range(128 // L):
                    x1 = soa_v[1, pl.ds(r10 * 128 + 16 * k, L)]
                    x2 = soa_v[2, pl.ds(r10 * 128 + 16 * k, L)]
                    y1 = soa_v[3, pl.ds(r10 * 128 + 16 * k, L)]
                    y2 = soa_v[4, pl.ds(r10 * 128 + 16 * k, L)]
                    ap = soa_v[5, pl.ds(r10 * 128 + 16 * k, L)]
                    ix1 = jnp.maximum(x1, t_x1)
                    iy1 = jnp.maximum(y1, t_y1)
                    ix2 = jnp.minimum(x2, t_x2)
                    iy2 = jnp.minimum(y2, t_y2)
                    inter = jnp.maximum(0.0, ix2 - ix1) * jnp.maximum(0.0, iy2 - iy1)
                    iou = inter / (ap + area_t - inter + 1e-06)
                    upd = iou > run_max
                    run_max = jnp.where(upd, iou, run_max)
                    run_idx = jnp.where(upd, idxs + 16 * k, run_idx)
                return run_max, run_idx, idxs + 128

            init = (jnp.full((L,), -jnp.inf, jnp.float32),
                    jnp.zeros((L,), jnp.int32), base_idx)
            run_max, run_idx, _ = lax.fori_loop(0, RPT, per_row, init)

            m = run_max
            for sh in (8, 4, 2, 1):
                m = jnp.maximum(m, m[iota ^ sh])
            cand = jnp.where(run_max == m, run_idx.astype(jnp.float32),
                             jnp.full((L,), 1e9, jnp.float32))
            for sh in (8, 4, 2, 1):
                cand = jnp.minimum(cand, cand[iota ^ sh])

            grp = lax.shift_right_logical(g, 4)
            posv = jnp.full((L,), g & (L - 1))
            resm_v[grp] = jnp.where(iota == posv, m, resm_v[grp])
            resi_v[grp] = jnp.where(iota == posv, cand, resi_v[grp])
            return carry

        lax.fori_loop(0, len_b, per_target, 0)

    pltpu.sync_copy(resm_v, shrm.at[s])
    pltpu.sync_copy(resi_v, shri.at[s])
    plsc.subcore_barrier()
    pltpu.sync_copy(shrm, mrgm_v)
    pltpu.sync_copy(shri, mrgi_v)

    out_v[...] = jnp.zeros((L,), jnp.float32)
    lane0 = iota == 0
    for gi in range(2):
        grp = s + NS * gi

        @pl.when(grp < NG)
        def _():
            mm = jnp.full((L,), -jnp.inf, jnp.float32)
            mi = jnp.zeros((L,), jnp.float32)
            for tt in range(NS):
                vm = mrgm_v[tt, grp]
                vi = mrgi_v[tt, grp]
                upd = vm > mm
                mm = jnp.where(upd, vm, mm)
                mi = jnp.where(upd, vi, mi)
            for po in range(L):
                g = grp * L + po
                b = g // T
                t = g - T * b
                scr_v[...] = lv[jnp.full((L,), b)]
                len_b = scr_v[...][0]
                valid = (t < len_b) & ((b & 1) == c)

                @pl.when(valid)
                def _():
                    best = jnp.where(mm[po] > 0.0, mi[po], 0.0).astype(jnp.int32)
                    w16 = best & ~(L - 1)
                    cps = [pltpu.async_copy(
                               relay_hbm.at[pl.ds(pl.multiple_of(
                                   (b * NF + f) * NPAD + w16, 8), L)],
                               row5_v.at[f], sem)
                           for f in range(5)]
                    for cp in cps:
                        cp.wait()
                    lane = jnp.full((L,), best & (L - 1))
                    ov = row5_v[0][lane]
                    x1v = row5_v[1][lane]
                    x2v = row5_v[2][lane]
                    y1v = row5_v[3][lane]
                    y2v = row5_v[4][lane]
                    vals = (ov, (x1v + x2v) * 0.5, (y1v + y2v) * 0.5,
                            x2v - x1v, y2v - y1v)
                    trow = tgt_v[g]
                    acc = out_v[...]
                    for r in range(5):
                        d = vals[r] - jnp.full((L,), trow[r])
                        acc = acc + jnp.where(lane0, d * d, 0.0)
                    out_v[...] = acc

    pltpu.sync_copy(out_v, out_hbm.at[wid])


def kernel(predictions, targets, lengths):
    B, N, F = predictions.shape
    T = targets.shape[1]
    NG = (B * T + L - 1) // L

    xp = jnp.transpose(predictions, (2, 0, 1))
    relay = pl.pallas_call(
        functools.partial(_tc_relayout, N, B),
        out_shape=jax.ShapeDtypeStruct((B * NF * RPF, 128), jnp.float32),
    )(xp)

    tgt_pad = jnp.reshape(jnp.pad(targets, ((0, 0), (0, 0), (0, L - F))),
                          (B * T, L))
    len_pad = jnp.pad(lengths.astype(jnp.int32), (0, L - B))

    mesh = plsc.VectorSubcoreMesh(core_axis_name="c", subcore_axis_name="s",
                                  num_cores=NC, num_subcores=NS)
    run = pl.kernel(
        functools.partial(_sc_loss, B, T),
        out_type=jax.ShapeDtypeStruct((NW, L), jnp.float32),
        mesh=mesh,
        compiler_params=pltpu.CompilerParams(use_tc_tiling_on_sc=False),
        scratch_types=[
            pltpu.VMEM((NF, RPT * 128), jnp.float32),
            pltpu.VMEM((B * T, L), jnp.float32),
            pltpu.VMEM((L,), jnp.int32),
            pltpu.VMEM((NG, L), jnp.float32),
            pltpu.VMEM((NG, L), jnp.float32),
            pltpu.VMEM_SHARED((NS, NG, L), jnp.float32),
            pltpu.VMEM_SHARED((NS, NG, L), jnp.float32),
            pltpu.VMEM((NS, NG, L), jnp.float32),
            pltpu.VMEM((NS, NG, L), jnp.float32),
            pltpu.VMEM((5, L), jnp.float32),
            pltpu.VMEM((L,), jnp.float32),
            pltpu.VMEM((L,), jnp.int32),
            pltpu.SemaphoreType.DMA,
        ],
    )
    partial = run(jnp.reshape(relay, (-1,)), tgt_pad, len_pad)
    return jnp.sum(partial) / B
